# Initial kernel scaffold; baseline (speedup 1.0000x reference)
import jax
import jax.numpy as jnp
from jax.experimental import pallas as pl


def kernel(hidden_states, Wq, bq, Wk, bk, Wvq, bvq, v_keys, v_embed, Wo, bo):
    raise NotImplementedError("write your pallas kernel here")


# speedup vs baseline: 1.1551x; 1.1551x over previous
import functools

import jax
import jax.numpy as jnp
from jax import lax
from jax.experimental import pallas as pl
from jax.experimental.pallas import tpu as pltpu
from jax.experimental.pallas import tpu_sc as plsc

B, S, D, H, NV, DR = 1, 2048, 1024, 16, 8192, 64
DH = D // H
SB = 256
NSB = S // SB
NEG = -1e30


def _proj_body(x_ref, wq_ref, bq_ref, wk_ref, bk_ref, wvq_ref, bvq_ref,
               vkt_ref, q_ref, k_ref, idx_ref):
    x = x_ref[...]
    q_ref[...] = jnp.dot(x, wq_ref[...],
                         preferred_element_type=jnp.float32) + bq_ref[...]
    k_ref[...] = jnp.dot(x, wk_ref[...],
                         preferred_element_type=jnp.float32) + bk_ref[...]
    vq = jnp.dot(x, wvq_ref[...],
                 preferred_element_type=jnp.float32) + bvq_ref[...]
    sim = jnp.dot(vq, vkt_ref[...], preferred_element_type=jnp.float32)
    m = jnp.max(sim, axis=1, keepdims=True)
    col = lax.broadcasted_iota(jnp.int32, sim.shape, 1)
    cand = jnp.where(sim == m, col, NV)
    idx_ref[0, 0, :] = jnp.min(cand, axis=1)


def _projections(x, Wq, bq, Wk, bk, Wvq, bvq, v_keys_t):
    full = lambda shape: pl.BlockSpec(shape, lambda i: (0,) * len(shape))
    return pl.pallas_call(
        _proj_body,
        grid=(NSB,),
        in_specs=[
            pl.BlockSpec((SB, D), lambda i: (i, 0)),
            full((D, D)), full((1, D)),
            full((D, D)), full((1, D)),
            full((D, DR)), full((1, DR)),
            full((DR, NV)),
        ],
        out_specs=[
            pl.BlockSpec((SB, D), lambda i: (i, 0)),
            pl.BlockSpec((SB, D), lambda i: (i, 0)),
            pl.BlockSpec((1, 1, SB), lambda i: (i, 0, 0)),
        ],
        out_shape=[
            jax.ShapeDtypeStruct((S, D), jnp.float32),
            jax.ShapeDtypeStruct((S, D), jnp.float32),
            jax.ShapeDtypeStruct((NSB, 1, SB), jnp.int32),
        ],
    )(x, Wq, bq.reshape(1, D), Wk, bk.reshape(1, D),
      Wvq, bvq.reshape(1, DR), v_keys_t)


_NW = 32
_BPW = S // _NW


def _sc_gather(idx, table):
    mesh = plsc.VectorSubcoreMesh(core_axis_name="c", subcore_axis_name="s")

    @functools.partial(
        pl.kernel, mesh=mesh,
        out_type=jax.ShapeDtypeStruct((S, D), jnp.float32),
        scratch_types=[
            pltpu.VMEM((_BPW,), jnp.int32),
            pltpu.VMEM((_BPW, D), jnp.float32),
            pltpu.SemaphoreType.DMA,
        ],
    )
    def k(idx_hbm, table_hbm, out_hbm, idx_v, rows_v, sem):
        wid = lax.axis_index("s") * 2 + lax.axis_index("c")
        base = wid * _BPW
        pltpu.sync_copy(idx_hbm.at[pl.ds(base, _BPW)], idx_v)
        pltpu.async_copy(table_hbm.at[idx_v], rows_v, sem).wait()
        pltpu.sync_copy(rows_v, out_hbm.at[pl.ds(base, _BPW)])

    return k(idx, table)


def _attn_body(q_ref, k_ref, x_ref, vs_ref, o_ref):
    q = q_ref[0] * 0.125
    s = lax.dot_general(q, k_ref[0], (((1,), (1,)), ((), ())),
                        preferred_element_type=jnp.float32)
    row = lax.broadcasted_iota(jnp.int32, (S, S), 0)
    col = lax.broadcasted_iota(jnp.int32, (S, S), 1)
    s = jnp.where(col <= row, s, NEG)
    m = jnp.max(s, axis=1, keepdims=True)
    p = jnp.exp(s - m)
    p = p / jnp.sum(p, axis=1, keepdims=True)
    v = x_ref[0] * vs_ref[0]
    o_ref[0] = jnp.dot(p, v, preferred_element_type=jnp.float32)


def _attention(q, k, x, v_sel):
    head = pl.BlockSpec((1, S, DH), lambda h: (h, 0, 0))
    return pl.pallas_call(
        _attn_body,
        grid=(H,),
        in_specs=[head, head, head, head],
        out_specs=head,
        out_shape=jax.ShapeDtypeStruct((H, S, DH), jnp.float32),
    )(q, k, x, v_sel)


def _out_body(a_ref, wo_ref, bo_ref, o_ref):
    o_ref[...] = jnp.dot(a_ref[...], wo_ref[...],
                         preferred_element_type=jnp.float32) + bo_ref[...]


def _out_proj(attn, Wo, bo):
    return pl.pallas_call(
        _out_body,
        out_shape=jax.ShapeDtypeStruct((S, D), jnp.float32),
    )(attn, Wo, bo.reshape(1, D))


def _heads(a):
    return a.reshape(S, H, DH).transpose(1, 0, 2)


def kernel(hidden_states, Wq, bq, Wk, bk, Wvq, bvq, v_keys, v_embed, Wo, bo):
    x = hidden_states.reshape(S, D)
    q, k, idx3 = _projections(x, Wq, bq, Wk, bk, Wvq, bvq, v_keys.T)
    v_sel = _sc_gather(idx3.reshape(S), v_embed)
    attn = _attention(_heads(q), _heads(k), _heads(x), _heads(v_sel))
    attn = attn.transpose(1, 0, 2).reshape(S, D)
    out = _out_proj(attn, Wo, bo)
    return out.reshape(B, S, D)


# speedup vs baseline: 1.7704x; 1.5327x over previous
import functools

import jax
import jax.numpy as jnp
from jax import lax
from jax.experimental import pallas as pl
from jax.experimental.pallas import tpu as pltpu
from jax.experimental.pallas import tpu_sc as plsc

B, S, D, H, NV, DR = 1, 2048, 1024, 16, 8192, 64
DH = D // H
SB = 256
NSB = S // SB
NEG = -1e30


def _proj_body(x_ref, wq_ref, bq_ref, wk_ref, bk_ref, wvq_ref, bvq_ref,
               vkt_ref, q_ref, k_ref, idx_ref):
    x = x_ref[...]
    q_ref[...] = jnp.dot(x, wq_ref[...],
                         preferred_element_type=jnp.float32) + bq_ref[...]
    k_ref[...] = jnp.dot(x, wk_ref[...],
                         preferred_element_type=jnp.float32) + bk_ref[...]
    vq = jnp.dot(x, wvq_ref[...],
                 preferred_element_type=jnp.float32) + bvq_ref[...]
    sim = jnp.dot(vq, vkt_ref[...], preferred_element_type=jnp.float32)
    m = jnp.max(sim, axis=1, keepdims=True)
    col = lax.broadcasted_iota(jnp.int32, sim.shape, 1)
    cand = jnp.where(sim == m, col, NV)
    idx_ref[0, 0, :] = jnp.min(cand, axis=1)


def _projections(x, Wq, bq, Wk, bk, Wvq, bvq, v_keys_t):
    full = lambda shape: pl.BlockSpec(shape, lambda i: (0,) * len(shape))
    return pl.pallas_call(
        _proj_body,
        grid=(NSB,),
        in_specs=[
            pl.BlockSpec((SB, D), lambda i: (i, 0)),
            full((D, D)), full((1, D)),
            full((D, D)), full((1, D)),
            full((D, DR)), full((1, DR)),
            full((DR, NV)),
        ],
        out_specs=[
            pl.BlockSpec((SB, D), lambda i: (i, 0)),
            pl.BlockSpec((SB, D), lambda i: (i, 0)),
            pl.BlockSpec((1, 1, SB), lambda i: (i, 0, 0)),
        ],
        out_shape=[
            jax.ShapeDtypeStruct((S, D), jnp.float32),
            jax.ShapeDtypeStruct((S, D), jnp.float32),
            jax.ShapeDtypeStruct((NSB, 1, SB), jnp.int32),
        ],
    )(x, Wq, bq.reshape(1, D), Wk, bk.reshape(1, D),
      Wvq, bvq.reshape(1, DR), v_keys_t)


_NW = 32
_BPW = S // _NW


def _sc_gather(idx, table):
    mesh = plsc.VectorSubcoreMesh(core_axis_name="c", subcore_axis_name="s")

    @functools.partial(
        pl.kernel, mesh=mesh,
        out_type=jax.ShapeDtypeStruct((S, D), jnp.float32),
        scratch_types=[
            pltpu.VMEM((_BPW,), jnp.int32),
            pltpu.VMEM((_BPW, D), jnp.float32),
            pltpu.SemaphoreType.DMA,
        ],
    )
    def k(idx_hbm, table_hbm, out_hbm, idx_v, rows_v, sem):
        wid = lax.axis_index("s") * 2 + lax.axis_index("c")
        base = wid * _BPW
        pltpu.sync_copy(idx_hbm.at[pl.ds(base, _BPW)], idx_v)
        pltpu.async_copy(table_hbm.at[idx_v], rows_v, sem).wait()
        pltpu.sync_copy(rows_v, out_hbm.at[pl.ds(base, _BPW)])

    return k(idx, table)


def _attn_body(q_ref, k_ref, x_ref, vs_ref, o_ref):
    row = lax.broadcasted_iota(jnp.int32, (S, S), 0)
    col = lax.broadcasted_iota(jnp.int32, (S, S), 1)
    causal = col <= row
    for j in range(2):
        sl = slice(j * DH, (j + 1) * DH)
        q = q_ref[:, sl] * 0.125
        s = lax.dot_general(q, k_ref[:, sl], (((1,), (1,)), ((), ())),
                            preferred_element_type=jnp.float32)
        s = jnp.where(causal, s, NEG)
        m = jnp.max(s, axis=1, keepdims=True)
        p = jnp.exp(s - m)
        p = p / jnp.sum(p, axis=1, keepdims=True)
        v = x_ref[:, sl] * vs_ref[:, sl]
        o_ref[:, sl] = jnp.dot(p, v, preferred_element_type=jnp.float32)


def _attention(q, k, x, v_sel):
    pair = pl.BlockSpec((S, 2 * DH), lambda h: (0, h))
    return pl.pallas_call(
        _attn_body,
        grid=(H // 2,),
        in_specs=[pair, pair, pair, pair],
        out_specs=pair,
        out_shape=jax.ShapeDtypeStruct((S, D), jnp.float32),
    )(q, k, x, v_sel)


def _out_body(a_ref, wo_ref, bo_ref, o_ref):
    o_ref[...] = jnp.dot(a_ref[...], wo_ref[...],
                         preferred_element_type=jnp.float32) + bo_ref[...]


def _out_proj(attn, Wo, bo):
    return pl.pallas_call(
        _out_body,
        out_shape=jax.ShapeDtypeStruct((S, D), jnp.float32),
    )(attn, Wo, bo.reshape(1, D))


def kernel(hidden_states, Wq, bq, Wk, bk, Wvq, bvq, v_keys, v_embed, Wo, bo):
    x = hidden_states.reshape(S, D)
    q, k, idx3 = _projections(x, Wq, bq, Wk, bk, Wvq, bvq, v_keys.T)
    v_sel = _sc_gather(idx3.reshape(S), v_embed)
    attn = _attention(q, k, x, v_sel)
    out = _out_proj(attn, Wo, bo)
    return out.reshape(B, S, D)
